# Initial kernel scaffold; baseline (speedup 1.0000x reference)
import jax
import jax.numpy as jnp
from jax.experimental import pallas as pl


def kernel(x, adj, eps, conv0_w, conv0_b, fc1_w, fc1_b, fc2_w, fc2_b, fc31_w, fc31_b, fc21_w, fc21_b, fc22_w, fc22_b, fc3_w, fc3_b, fc32_w, fc32_b, fc4_w, fc4_b, conv1_w, conv1_b, g1_w, g3_w, g4_w, g5_w, fcc_w, fcc_b):
    raise NotImplementedError("write your pallas kernel here")


# speedup vs baseline: 1.1632x; 1.1632x over previous
import functools

import jax
import jax.numpy as jnp
from jax.experimental import pallas as pl

N = 10000
VAR = 4
NIN = 256
NZ = 100
NC = 10

_BM_AE = 1000
_BM_G = 400


def _mm(a, b):
    return jax.lax.dot_general(a, b, (((1,), (0,)), ((), ())),
                               preferred_element_type=jnp.float32)


def _shift_right(v):
    return jnp.concatenate([jnp.zeros((v.shape[0], 1), v.dtype), v[:, :-1]], axis=1)


def _shift_left(v):
    return jnp.concatenate([v[:, 1:], jnp.zeros((v.shape[0], 1), v.dtype)], axis=1)


def _ae_body(x_ref, eps_ref, c0w_ref, c0b_ref, c1w_ref, c1b_ref,
             f1w_ref, f1b_ref, f2w_ref, f2b_ref, f31w_ref, f31b_ref,
             f21w_ref, f21b_ref, f22w_ref, f22b_ref,
             f3w_ref, f3b_ref, f32w_ref, f32b_ref, f4w_ref, f4b_ref,
             g1w_ref,
             out0_ref, mu_ref, logvar_ref, s1_ref):
    x = x_ref[...]
    pro = jnp.broadcast_to(c0b_ref[0:1, 0:1], (x.shape[0], NIN)).astype(jnp.float32)
    for c in range(VAR):
        xc = x[:, c * NIN:(c + 1) * NIN]
        pro = pro + c0w_ref[c:c + 1, 0:1] * _shift_right(xc)
        pro = pro + c0w_ref[c:c + 1, 1:2] * xc
        pro = pro + c0w_ref[c:c + 1, 2:3] * _shift_left(xc)
    h1 = jax.nn.relu(_mm(pro, f1w_ref[...]) + f1b_ref[...])
    h2 = jax.nn.relu(_mm(h1, f2w_ref[...]) + f2b_ref[...])
    h3 = jax.nn.relu(_mm(h2, f31w_ref[...]) + f31b_ref[...])
    mu = _mm(h3, f21w_ref[...]) + f21b_ref[...]
    logvar = _mm(h3, f22w_ref[...]) + f22b_ref[...]
    std = jnp.exp(0.5 * logvar)
    z = eps_ref[...] * std + mu
    d3 = jax.nn.relu(_mm(z, f3w_ref[...]) + f3b_ref[...])
    d4 = jax.nn.relu(_mm(d3, f32w_ref[...]) + f32b_ref[...])
    recon = jax.nn.sigmoid(_mm(d4, f4w_ref[...]) + f4b_ref[...])
    for co in range(VAR):
        o = c1b_ref[0:1, co:co + 1] + c1w_ref[co:co + 1, 1:2] * recon
        o = o + c1w_ref[co:co + 1, 0:1] * _shift_right(recon)
        o = o + c1w_ref[co:co + 1, 2:3] * _shift_left(recon)
        out0_ref[:, co * NIN:(co + 1) * NIN] = o
    mu_ref[...] = mu
    logvar_ref[...] = logvar
    s1_ref[...] = _mm(pro, g1w_ref[...])


def _gnn_body(adj_ref, s_ref, w_ref, b_ref, out_ref, *, act, last):
    h = _mm(adj_ref[...], s_ref[...])
    if act:
        h = jax.nn.relu(h)
    y = _mm(h, w_ref[...])
    if last:
        logits = y + b_ref[...]
        m = jnp.max(logits, axis=1, keepdims=True)
        e = jnp.exp(logits - m)
        out_ref[...] = e / jnp.sum(e, axis=1, keepdims=True)
    else:
        out_ref[...] = y


def _full_spec(shape):
    nd = len(shape)
    return pl.BlockSpec(shape, lambda i, _n=nd: (0,) * _n)


def _gnn_layer(adj, s, w, b, *, act, last):
    nb = N // _BM_G
    out_cols = NC if last else s.shape[1]
    body = functools.partial(_gnn_body, act=act, last=last)
    return pl.pallas_call(
        body,
        grid=(nb,),
        in_specs=[
            pl.BlockSpec((_BM_G, N), lambda i: (i, 0)),
            _full_spec(s.shape),
            _full_spec(w.shape),
            _full_spec(b.shape),
        ],
        out_specs=pl.BlockSpec((_BM_G, out_cols), lambda i: (i, 0)),
        out_shape=jax.ShapeDtypeStruct((N, out_cols), jnp.float32),
    )(adj, s, w, b)


def kernel(x, adj, eps, conv0_w, conv0_b, fc1_w, fc1_b, fc2_w, fc2_b,
           fc31_w, fc31_b, fc21_w, fc21_b, fc22_w, fc22_b, fc3_w, fc3_b,
           fc32_w, fc32_b, fc4_w, fc4_b, conv1_w, conv1_b,
           g1_w, g3_w, g4_w, g5_w, fcc_w, fcc_b):
    f32 = jnp.float32
    x2 = x.reshape(N, VAR * NIN)
    c0w = conv0_w.reshape(VAR, 3)
    c0b = conv0_b.reshape(1, 1)
    c1w = conv1_w.reshape(VAR, 3)
    c1b = conv1_b.reshape(1, VAR)
    biases = dict(
        f1b=fc1_b.reshape(1, -1), f2b=fc2_b.reshape(1, -1),
        f31b=fc31_b.reshape(1, -1), f21b=fc21_b.reshape(1, -1),
        f22b=fc22_b.reshape(1, -1), f3b=fc3_b.reshape(1, -1),
        f32b=fc32_b.reshape(1, -1), f4b=fc4_b.reshape(1, -1),
    )

    nb = N // _BM_AE
    ae_inputs = (x2, eps, c0w, c0b, c1w, c1b,
                 fc1_w, biases['f1b'], fc2_w, biases['f2b'],
                 fc31_w, biases['f31b'], fc21_w, biases['f21b'],
                 fc22_w, biases['f22b'], fc3_w, biases['f3b'],
                 fc32_w, biases['f32b'], fc4_w, biases['f4b'], g1_w)
    in_specs = [
        pl.BlockSpec((_BM_AE, VAR * NIN), lambda i: (i, 0)),
        pl.BlockSpec((_BM_AE, NZ), lambda i: (i, 0)),
    ] + [_full_spec(a.shape) for a in ae_inputs[2:]]
    out_specs = [
        pl.BlockSpec((_BM_AE, VAR * NIN), lambda i: (i, 0)),
        pl.BlockSpec((_BM_AE, NZ), lambda i: (i, 0)),
        pl.BlockSpec((_BM_AE, NZ), lambda i: (i, 0)),
        pl.BlockSpec((_BM_AE, NZ), lambda i: (i, 0)),
    ]
    out_shape = [
        jax.ShapeDtypeStruct((N, VAR * NIN), f32),
        jax.ShapeDtypeStruct((N, NZ), f32),
        jax.ShapeDtypeStruct((N, NZ), f32),
        jax.ShapeDtypeStruct((N, NZ), f32),
    ]
    out0_flat, mu, logvar, s1 = pl.pallas_call(
        _ae_body,
        grid=(nb,),
        in_specs=in_specs,
        out_specs=out_specs,
        out_shape=out_shape,
    )(*ae_inputs)

    dummy_b = jnp.zeros((1, 1), f32)
    s2 = _gnn_layer(adj, s1, g3_w, dummy_b, act=True, last=False)
    s3 = _gnn_layer(adj, s2, g4_w, dummy_b, act=True, last=False)
    s4 = _gnn_layer(adj, s3, g5_w, dummy_b, act=False, last=False)
    predict = _gnn_layer(adj, s4, fcc_w, fcc_b.reshape(1, NC), act=False,
                         last=True)

    out0 = out0_flat.reshape(N, VAR, NIN)
    return (out0, predict, mu, logvar)


# speedup vs baseline: 1.2400x; 1.0660x over previous
import functools

import jax
import jax.numpy as jnp
from jax.experimental import pallas as pl

N = 10000
VAR = 4
NIN = 256
NZ = 100
NC = 10

_BM_AE = 1000
_BM_G = 400


def _mm(a, b):
    return jax.lax.dot_general(a, b, (((1,), (0,)), ((), ())),
                               preferred_element_type=jnp.float32)


def _shift_right(v):
    return jnp.concatenate([jnp.zeros((v.shape[0], 1), v.dtype), v[:, :-1]], axis=1)


def _shift_left(v):
    return jnp.concatenate([v[:, 1:], jnp.zeros((v.shape[0], 1), v.dtype)], axis=1)


def _ae_body(x_ref, eps_ref, c0w_ref, c0b_ref, c1w_ref, c1b_ref,
             f1w_ref, f1b_ref, f2w_ref, f2b_ref, f31w_ref, f31b_ref,
             f21w_ref, f21b_ref, f22w_ref, f22b_ref,
             f3w_ref, f3b_ref, f32w_ref, f32b_ref, f4w_ref, f4b_ref,
             g1w_ref,
             out0_ref, mu_ref, logvar_ref, s1_ref):
    x = x_ref[...]
    pro = jnp.broadcast_to(c0b_ref[0:1, 0:1], (x.shape[0], NIN)).astype(jnp.float32)
    for c in range(VAR):
        xc = x[:, c, :]
        pro = pro + c0w_ref[c:c + 1, 0:1] * _shift_right(xc)
        pro = pro + c0w_ref[c:c + 1, 1:2] * xc
        pro = pro + c0w_ref[c:c + 1, 2:3] * _shift_left(xc)
    h1 = jax.nn.relu(_mm(pro, f1w_ref[...]) + f1b_ref[...])
    h2 = jax.nn.relu(_mm(h1, f2w_ref[...]) + f2b_ref[...])
    h3 = jax.nn.relu(_mm(h2, f31w_ref[...]) + f31b_ref[...])
    mu = _mm(h3, f21w_ref[...]) + f21b_ref[...]
    logvar = _mm(h3, f22w_ref[...]) + f22b_ref[...]
    std = jnp.exp(0.5 * logvar)
    z = eps_ref[...] * std + mu
    d3 = jax.nn.relu(_mm(z, f3w_ref[...]) + f3b_ref[...])
    d4 = jax.nn.relu(_mm(d3, f32w_ref[...]) + f32b_ref[...])
    recon = jax.nn.sigmoid(_mm(d4, f4w_ref[...]) + f4b_ref[...])
    for co in range(VAR):
        o = c1b_ref[0:1, co:co + 1] + c1w_ref[co:co + 1, 1:2] * recon
        o = o + c1w_ref[co:co + 1, 0:1] * _shift_right(recon)
        o = o + c1w_ref[co:co + 1, 2:3] * _shift_left(recon)
        out0_ref[:, co, :] = o
    mu_ref[...] = mu
    logvar_ref[...] = logvar
    s1_ref[...] = _mm(pro, g1w_ref[...])


def _gnn_body(adj_ref, s_ref, w_ref, b_ref, out_ref, *, act, last):
    h = _mm(adj_ref[...], s_ref[...])
    if act:
        h = jax.nn.relu(h)
    y = _mm(h, w_ref[...])
    if last:
        logits = y + b_ref[...]
        m = jnp.max(logits, axis=1, keepdims=True)
        e = jnp.exp(logits - m)
        out_ref[...] = e / jnp.sum(e, axis=1, keepdims=True)
    else:
        out_ref[...] = y


def _full_spec(shape):
    nd = len(shape)
    return pl.BlockSpec(shape, lambda i, _n=nd: (0,) * _n)


def _gnn_layer(adj, s, w, b, *, act, last):
    nb = N // _BM_G
    out_cols = NC if last else s.shape[1]
    body = functools.partial(_gnn_body, act=act, last=last)
    return pl.pallas_call(
        body,
        grid=(nb,),
        in_specs=[
            pl.BlockSpec((_BM_G, N), lambda i: (i, 0)),
            _full_spec(s.shape),
            _full_spec(w.shape),
            _full_spec(b.shape),
        ],
        out_specs=pl.BlockSpec((_BM_G, out_cols), lambda i: (i, 0)),
        out_shape=jax.ShapeDtypeStruct((N, out_cols), jnp.float32),
    )(adj, s, w, b)


def kernel(x, adj, eps, conv0_w, conv0_b, fc1_w, fc1_b, fc2_w, fc2_b,
           fc31_w, fc31_b, fc21_w, fc21_b, fc22_w, fc22_b, fc3_w, fc3_b,
           fc32_w, fc32_b, fc4_w, fc4_b, conv1_w, conv1_b,
           g1_w, g3_w, g4_w, g5_w, fcc_w, fcc_b):
    f32 = jnp.float32
    c0w = conv0_w.reshape(VAR, 3)
    c0b = conv0_b.reshape(1, 1)
    c1w = conv1_w.reshape(VAR, 3)
    c1b = conv1_b.reshape(1, VAR)
    biases = dict(
        f1b=fc1_b.reshape(1, -1), f2b=fc2_b.reshape(1, -1),
        f31b=fc31_b.reshape(1, -1), f21b=fc21_b.reshape(1, -1),
        f22b=fc22_b.reshape(1, -1), f3b=fc3_b.reshape(1, -1),
        f32b=fc32_b.reshape(1, -1), f4b=fc4_b.reshape(1, -1),
    )

    nb = N // _BM_AE
    ae_inputs = (x, eps, c0w, c0b, c1w, c1b,
                 fc1_w, biases['f1b'], fc2_w, biases['f2b'],
                 fc31_w, biases['f31b'], fc21_w, biases['f21b'],
                 fc22_w, biases['f22b'], fc3_w, biases['f3b'],
                 fc32_w, biases['f32b'], fc4_w, biases['f4b'], g1_w)
    in_specs = [
        pl.BlockSpec((_BM_AE, VAR, NIN), lambda i: (i, 0, 0)),
        pl.BlockSpec((_BM_AE, NZ), lambda i: (i, 0)),
    ] + [_full_spec(a.shape) for a in ae_inputs[2:]]
    out_specs = [
        pl.BlockSpec((_BM_AE, VAR, NIN), lambda i: (i, 0, 0)),
        pl.BlockSpec((_BM_AE, NZ), lambda i: (i, 0)),
        pl.BlockSpec((_BM_AE, NZ), lambda i: (i, 0)),
        pl.BlockSpec((_BM_AE, NZ), lambda i: (i, 0)),
    ]
    out_shape = [
        jax.ShapeDtypeStruct((N, VAR, NIN), f32),
        jax.ShapeDtypeStruct((N, NZ), f32),
        jax.ShapeDtypeStruct((N, NZ), f32),
        jax.ShapeDtypeStruct((N, NZ), f32),
    ]
    out0, mu, logvar, s1 = pl.pallas_call(
        _ae_body,
        grid=(nb,),
        in_specs=in_specs,
        out_specs=out_specs,
        out_shape=out_shape,
    )(*ae_inputs)

    dummy_b = jnp.zeros((1, 1), f32)
    s2 = _gnn_layer(adj, s1, g3_w, dummy_b, act=True, last=False)
    s3 = _gnn_layer(adj, s2, g4_w, dummy_b, act=True, last=False)
    s4 = _gnn_layer(adj, s3, g5_w, dummy_b, act=False, last=False)
    predict = _gnn_layer(adj, s4, fcc_w, fcc_b.reshape(1, NC), act=False,
                         last=True)

    return (out0, predict, mu, logvar)


# speedup vs baseline: 1.3337x; 1.0756x over previous
import functools

import jax
import jax.numpy as jnp
from jax.experimental import pallas as pl

N = 10000
VAR = 4
NIN = 256
NZ = 100
NC = 10

_BM_AE = 1000
_BM_G = 400


def _mm(a, b):
    return jax.lax.dot_general(a, b, (((1,), (0,)), ((), ())),
                               preferred_element_type=jnp.float32)


def _ae_body(x_ref, eps_ref, t0_ref, c0b_ref, t1_ref, c1b_ref,
             f1w_ref, f1b_ref, f2w_ref, f2b_ref, f31w_ref, f31b_ref,
             f21w_ref, f21b_ref, f22w_ref, f22b_ref,
             f3w_ref, f3b_ref, f32w_ref, f32b_ref, f4w_ref, f4b_ref,
             g1w_ref,
             out0_ref, mu_ref, logvar_ref, s1_ref):
    x = x_ref[...]
    pc = jax.lax.dot_general(x, t0_ref[...], (((2,), (1,)), ((1,), (0,))),
                             preferred_element_type=jnp.float32)
    pro = jnp.sum(pc, axis=0) + c0b_ref[0:1, 0:1]
    h1 = jax.nn.relu(_mm(pro, f1w_ref[...]) + f1b_ref[...])
    h2 = jax.nn.relu(_mm(h1, f2w_ref[...]) + f2b_ref[...])
    h3 = jax.nn.relu(_mm(h2, f31w_ref[...]) + f31b_ref[...])
    mu = _mm(h3, f21w_ref[...]) + f21b_ref[...]
    logvar = _mm(h3, f22w_ref[...]) + f22b_ref[...]
    std = jnp.exp(0.5 * logvar)
    z = eps_ref[...] * std + mu
    d3 = jax.nn.relu(_mm(z, f3w_ref[...]) + f3b_ref[...])
    d4 = jax.nn.relu(_mm(d3, f32w_ref[...]) + f32b_ref[...])
    recon = jax.nn.sigmoid(_mm(d4, f4w_ref[...]) + f4b_ref[...])
    for co in range(VAR):
        out0_ref[:, co, :] = (_mm(recon, t1_ref[co]) + c1b_ref[0:1, co:co + 1])
    mu_ref[...] = mu
    logvar_ref[...] = logvar
    s1_ref[...] = _mm(pro, g1w_ref[...])


def _gnn_body(adj_ref, s_ref, w_ref, b_ref, out_ref, *, act, last):
    h = _mm(adj_ref[...], s_ref[...])
    if act:
        h = jax.nn.relu(h)
    y = _mm(h, w_ref[...])
    if last:
        logits = y + b_ref[...]
        m = jnp.max(logits, axis=1, keepdims=True)
        e = jnp.exp(logits - m)
        out_ref[...] = e / jnp.sum(e, axis=1, keepdims=True)
    else:
        out_ref[...] = y


def _full_spec(shape):
    nd = len(shape)
    return pl.BlockSpec(shape, lambda i, _n=nd: (0,) * _n)


def _gnn_layer(adj, s, w, b, *, act, last):
    nb = N // _BM_G
    out_cols = NC if last else s.shape[1]
    body = functools.partial(_gnn_body, act=act, last=last)
    return pl.pallas_call(
        body,
        grid=(nb,),
        in_specs=[
            pl.BlockSpec((_BM_G, N), lambda i: (i, 0)),
            _full_spec(s.shape),
            _full_spec(w.shape),
            _full_spec(b.shape),
        ],
        out_specs=pl.BlockSpec((_BM_G, out_cols), lambda i: (i, 0)),
        out_shape=jax.ShapeDtypeStruct((N, out_cols), jnp.float32),
    )(adj, s, w, b)


def kernel(x, adj, eps, conv0_w, conv0_b, fc1_w, fc1_b, fc2_w, fc2_b,
           fc31_w, fc31_b, fc21_w, fc21_b, fc22_w, fc22_b, fc3_w, fc3_b,
           fc32_w, fc32_b, fc4_w, fc4_b, conv1_w, conv1_b,
           g1_w, g3_w, g4_w, g5_w, fcc_w, fcc_b):
    f32 = jnp.float32
    c0w = conv0_w.reshape(VAR, 3)
    c0b = conv0_b.reshape(1, 1)
    c1w = conv1_w.reshape(VAR, 3)
    c1b = conv1_b.reshape(1, VAR)
    e_up = jnp.eye(NIN, k=1, dtype=f32)
    e_d = jnp.eye(NIN, dtype=f32)
    e_dn = jnp.eye(NIN, k=-1, dtype=f32)
    t0 = (c0w[:, 0, None, None] * e_up + c0w[:, 1, None, None] * e_d
          + c0w[:, 2, None, None] * e_dn)
    t1 = (c1w[:, 0, None, None] * e_up + c1w[:, 1, None, None] * e_d
          + c1w[:, 2, None, None] * e_dn)
    biases = dict(
        f1b=fc1_b.reshape(1, -1), f2b=fc2_b.reshape(1, -1),
        f31b=fc31_b.reshape(1, -1), f21b=fc21_b.reshape(1, -1),
        f22b=fc22_b.reshape(1, -1), f3b=fc3_b.reshape(1, -1),
        f32b=fc32_b.reshape(1, -1), f4b=fc4_b.reshape(1, -1),
    )

    nb = N // _BM_AE
    ae_inputs = (x, eps, t0, c0b, t1, c1b,
                 fc1_w, biases['f1b'], fc2_w, biases['f2b'],
                 fc31_w, biases['f31b'], fc21_w, biases['f21b'],
                 fc22_w, biases['f22b'], fc3_w, biases['f3b'],
                 fc32_w, biases['f32b'], fc4_w, biases['f4b'], g1_w)
    in_specs = [
        pl.BlockSpec((_BM_AE, VAR, NIN), lambda i: (i, 0, 0)),
        pl.BlockSpec((_BM_AE, NZ), lambda i: (i, 0)),
    ] + [_full_spec(a.shape) for a in ae_inputs[2:]]
    out_specs = [
        pl.BlockSpec((_BM_AE, VAR, NIN), lambda i: (i, 0, 0)),
        pl.BlockSpec((_BM_AE, NZ), lambda i: (i, 0)),
        pl.BlockSpec((_BM_AE, NZ), lambda i: (i, 0)),
        pl.BlockSpec((_BM_AE, NZ), lambda i: (i, 0)),
    ]
    out_shape = [
        jax.ShapeDtypeStruct((N, VAR, NIN), f32),
        jax.ShapeDtypeStruct((N, NZ), f32),
        jax.ShapeDtypeStruct((N, NZ), f32),
        jax.ShapeDtypeStruct((N, NZ), f32),
    ]
    out0, mu, logvar, s1 = pl.pallas_call(
        _ae_body,
        grid=(nb,),
        in_specs=in_specs,
        out_specs=out_specs,
        out_shape=out_shape,
    )(*ae_inputs)

    dummy_b = jnp.zeros((1, 1), f32)
    s2 = _gnn_layer(adj, s1, g3_w, dummy_b, act=True, last=False)
    s3 = _gnn_layer(adj, s2, g4_w, dummy_b, act=True, last=False)
    s4 = _gnn_layer(adj, s3, g5_w, dummy_b, act=False, last=False)
    predict = _gnn_layer(adj, s4, fcc_w, fcc_b.reshape(1, NC), act=False,
                         last=True)

    return (out0, predict, mu, logvar)
